# baseline (speedup 1.0000x reference)
import jax
import jax.numpy as jnp
from jax import lax
from jax.experimental import pallas as pl
from jax.experimental.pallas import tpu as pltpu
from jax.experimental.pallas import tpu_sc as plsc

USER = 25000
ITEM = 25000
N = USER + ITEM
E = 800000
LATDIM = 64
HD = LATDIM // 2
RIS_ADJ_LAMBDA = 0.2
RIS_LAMBDA = 0.5

NSUB = 16
SPAN = E // NSUB
CHUNK = 80
MEGA = 2000
NK = MEGA // CHUNK
NM = SPAN // MEGA
ACC_ROWS = 50048
ZROWS = ACC_ROWS // NSUB


NBUF = 6
PRE = 3


def _spmm_body(rows_hbm, cols_hbm, vals_hbm, xs_hbm, zeros_hbm, out_hbm,
               acc, cols_v, vals_v,
               rows_a, rows_b, rows_c, rows_d, rows_e, rows_f,
               gath_a, gath_b, gath_c, gath_d, gath_e, gath_f,
               sem_s, sem_a, sem_b, sem_c, sem_d, sem_e, sem_f,
               sem_sa, sem_sb, sem_sc, sem_sd, sem_se, sem_sf):
    core = lax.axis_index("c")
    sid = lax.axis_index("s")

    RB = (rows_a, rows_b, rows_c, rows_d, rows_e, rows_f)
    GB = (gath_a, gath_b, gath_c, gath_d, gath_e, gath_f)
    SG = (sem_a, sem_b, sem_c, sem_d, sem_e, sem_f)
    SS = (sem_sa, sem_sb, sem_sc, sem_sd, sem_se, sem_sf)

    pltpu.sync_copy(zeros_hbm, acc.at[pl.ds(sid * ZROWS, ZROWS)])

    xsrc = xs_hbm.at[core]

    def start(cb, k, p):
        pltpu.async_copy(rows_hbm.at[cb + k], RB[p], SG[p])
        pltpu.async_copy(xsrc.at[cols_v.at[k]], GB[p], SG[p])

    def waitg(cb, k, p):
        pltpu.make_async_copy(rows_hbm.at[cb + k], RB[p], SG[p]).wait()
        pltpu.make_async_copy(xsrc.at[cols_v.at[k]], GB[p], SG[p]).wait()

    def compute(k, p):
        gath_buf = GB[p]
        off = k * CHUNK
        for j in range(CHUNK // 16):
            val16 = vals_v[pl.ds(off + j * 16, 16)]
            for i in range(16):
                e = j * 16 + i
                vv = lax.gather(
                    val16, jnp.full((16, 1), i, jnp.int32),
                    lax.GatherDimensionNumbers(
                        offset_dims=(), collapsed_slice_dims=(0,),
                        start_index_map=(0,)),
                    slice_sizes=(1,),
                    mode=lax.GatherScatterMode.PROMISE_IN_BOUNDS)
                for c in range(HD // 16):
                    sl = pl.ds(c * 16, 16)
                    gath_buf[e, sl] = gath_buf[e, sl] * vv

    def scat_start(p):
        pltpu.async_copy(GB[p], acc.at[RB[p]], SS[p], add=True)

    def scat_wait(p):
        pltpu.make_async_copy(GB[p], acc.at[RB[p]], SS[p]).wait()

    for p in range(NBUF):
        for e in range(CHUNK):
            for c in range(HD // 16):
                GB[p][e, pl.ds(c * 16, 16)] = jnp.zeros((16,), jnp.float32)
        for e in range(CHUNK // 16):
            RB[p][pl.ds(e * 16, 16)] = jnp.zeros((16,), jnp.int32)
    plsc.subcore_barrier()
    for p in range(NBUF):
        scat_start(p)

    def mega_body(m, _):
        base = sid * SPAN + m * MEGA
        cb = base // CHUNK
        d_cols = pltpu.async_copy(cols_hbm.at[pl.ds(cb, NK)], cols_v, sem_s)
        d_vals = pltpu.async_copy(vals_hbm.at[pl.ds(base, MEGA)], vals_v,
                                  sem_s)
        d_cols.wait()
        d_vals.wait()
        for k in range(PRE):
            scat_wait(k % NBUF)
            start(cb, k, k % NBUF)

        def quad_body(t, _):
            k0 = NBUF * t
            for p in range(NBUF):
                k = k0 + p
                waitg(cb, k, p)
                w = (p + PRE) % NBUF
                scat_wait(w)
                start(cb, k + PRE, w)
                compute(k, p)
                scat_start(p)
            return 0

        lax.fori_loop(0, (NK - 5) // NBUF, quad_body, 0)
        for k in range(((NK - 5) // NBUF) * NBUF, NK):
            p = k % NBUF
            waitg(cb, k, p)
            if k + PRE < NK:
                w = (k + PRE) % NBUF
                scat_wait(w)
                start(cb, k + PRE, w)
            compute(k, p)
            scat_start(p)
        return 0

    lax.fori_loop(0, NM, mega_body, 0)
    for p in range(NBUF):
        scat_wait(p)
    plsc.subcore_barrier()

    fl = 3120

    @pl.when(sid < 15)
    def _():
        pltpu.sync_copy(acc.at[pl.ds(sid * fl, fl)],
                        out_hbm.at[core, pl.ds(sid * fl, fl)])

    @pl.when(sid == 15)
    def _():
        pltpu.sync_copy(acc.at[pl.ds(15 * fl, N - 15 * fl)],
                        out_hbm.at[core, pl.ds(15 * fl, N - 15 * fl)])


_spmm_call = pl.kernel(
    _spmm_body,
    out_type=jax.ShapeDtypeStruct((2, N, HD), jnp.float32),
    mesh=plsc.VectorSubcoreMesh(core_axis_name="c", subcore_axis_name="s"),
    compiler_params=pltpu.CompilerParams(
        needs_layout_passes=False, use_tc_tiling_on_sc=False),
    scratch_types=[
        pltpu.VMEM_SHARED((ACC_ROWS, HD), jnp.float32),
        pltpu.VMEM((NK, CHUNK), jnp.int32),
        pltpu.VMEM((MEGA,), jnp.float32),
    ] + [pltpu.VMEM((CHUNK,), jnp.int32)] * 6
      + [pltpu.VMEM((CHUNK, HD), jnp.float32)] * 6
      + [pltpu.SemaphoreType.DMA] * 13,
)

_ZEROS = None


def _spmm(idx, val, xs):
    global _ZEROS
    if _ZEROS is None:
        _ZEROS = jnp.zeros((ZROWS, HD), jnp.float32)
    rows2 = idx[0].reshape(E // CHUNK, CHUNK)
    cols2 = idx[1].reshape(E // CHUNK, CHUNK)
    return _spmm_call(rows2, cols2, val, xs, _ZEROS)


def _split(x):
    return jnp.stack([x[:, :HD], x[:, HD:]])


_FEAT_BLK = 1000


def _feats_body(x_ref, w_ref, b_ref, o_ref):
    acc = jnp.dot(x_ref[...], w_ref[...], preferred_element_type=jnp.float32)
    acc = acc + b_ref[...]
    n = jnp.sqrt(jnp.sum(acc * acc, axis=1, keepdims=True))
    o_ref[...] = acc / jnp.maximum(n, 1e-12)


def _proj_l2(x, w, b):
    m, k = x.shape
    d = w.shape[1]
    grid = m // _FEAT_BLK
    return pl.pallas_call(
        _feats_body,
        grid=(grid,),
        in_specs=[
            pl.BlockSpec((_FEAT_BLK, k), lambda i: (i, 0)),
            pl.BlockSpec((k, d), lambda i: (0, 0)),
            pl.BlockSpec((1, d), lambda i: (0, 0)),
        ],
        out_specs=pl.BlockSpec((_FEAT_BLK, d), lambda i: (i, 0)),
        out_shape=jax.ShapeDtypeStruct((m, d), jnp.float32),
    )(x, w, b.reshape(1, d))


_FUSE_BLK = 1000


def _modal_body(ia, ib, iadj, ta, tb, tadj, wi, wt, o_ref):
    img = ia[...] + ib[...] + RIS_ADJ_LAMBDA * iadj[...]
    txt = ta[...] + tb[...] + RIS_ADJ_LAMBDA * tadj[...]
    o_ref[...] = wi[...] * img + wt[...] * txt


def _modal_combine(im1, im2, imadj, tx1, tx2, txadj, wi, wt):
    spec = pl.BlockSpec((2, _FUSE_BLK, HD), lambda i: (0, i, 0))
    wspec = pl.BlockSpec((_FUSE_BLK, 1), lambda i: (i, 0))
    return pl.pallas_call(
        _modal_body,
        grid=(N // _FUSE_BLK,),
        in_specs=[spec, spec, spec, spec, spec, spec, wspec, wspec],
        out_specs=spec,
        out_shape=jax.ShapeDtypeStruct((2, N, HD), jnp.float32),
    )(im1, im2, imadj, tx1, tx2, txadj, wi, wt)


def _final_body(m, g1, g2, o_ref):
    m0, m1 = m[0], m[1]
    s = jnp.sum(m0 * m0 + m1 * m1, axis=1, keepdims=True)
    inv = RIS_LAMBDA / jnp.maximum(jnp.sqrt(s), 1e-12)
    o_ref[:, :HD] = m0 + g1[0] + g2[0] + inv * m0
    o_ref[:, HD:] = m1 + g1[1] + g2[1] + inv * m1


def _final_combine(modal, g1, g2):
    spec = pl.BlockSpec((2, _FUSE_BLK, HD), lambda i: (0, i, 0))
    return pl.pallas_call(
        _final_body,
        grid=(N // _FUSE_BLK,),
        in_specs=[spec, spec, spec],
        out_specs=pl.BlockSpec((_FUSE_BLK, LATDIM), lambda i: (i, 0)),
        out_shape=jax.ShapeDtypeStruct((N, LATDIM), jnp.float32),
    )(modal, g1, g2)


def kernel(adj_idx, adj_val, image_adj_idx, image_adj_val, text_adj_idx,
           text_adj_val, att_image_list, att_text_list, uEmbeds, iEmbeds,
           image_embedding, text_embedding, Wi, bi, Wt, bt):
    image_feats_n = _proj_l2(image_embedding, Wi, bi)
    text_feats_n = _proj_l2(text_embedding, Wt, bt)

    u_s = _split(uEmbeds)
    i_s = _split(iEmbeds)
    ui_s = jnp.concatenate([u_s, i_s], axis=1)

    embedsImageAdj = _spmm(image_adj_idx, image_adj_val, ui_s)
    embedsTextAdj = _spmm(text_adj_idx, text_adj_val, ui_s)

    embedsImage1 = _spmm(adj_idx, adj_val,
                         jnp.concatenate([u_s, _split(image_feats_n)], axis=1))
    embedsImage2 = _spmm(adj_idx, adj_val,
                         jnp.concatenate([embedsImage1[:, :USER], i_s], axis=1))
    embedsText1 = _spmm(adj_idx, adj_val,
                        jnp.concatenate([u_s, _split(text_feats_n)], axis=1))
    embedsText2 = _spmm(adj_idx, adj_val,
                        jnp.concatenate([embedsText1[:, :USER], i_s], axis=1))

    weight_sum = att_image_list + att_text_list
    weight_sum = jnp.where(weight_sum == 0, jnp.ones_like(weight_sum), weight_sum)
    wi_att = (att_image_list / weight_sum)[:, None]
    wt_att = (att_text_list / weight_sum)[:, None]

    embedsModal = _modal_combine(embedsImage1, embedsImage2, embedsImageAdj,
                                 embedsText1, embedsText2, embedsTextAdj,
                                 wi_att, wt_att)

    g1 = _spmm(adj_idx, adj_val, embedsModal)
    g2 = _spmm(adj_idx, adj_val, g1)
    embeds = _final_combine(embedsModal, g1, g2)
    return (embeds[:USER], embeds[USER:])


# speedup vs baseline: 1.2623x; 1.2623x over previous
import jax
import jax.numpy as jnp
from jax import lax
from jax.experimental import pallas as pl
from jax.experimental.pallas import tpu as pltpu
from jax.experimental.pallas import tpu_sc as plsc

USER = 25000
ITEM = 25000
N = USER + ITEM
E = 800000
LATDIM = 64
HD = LATDIM // 2
RIS_ADJ_LAMBDA = 0.2
RIS_LAMBDA = 0.5

NSUB = 16
SPAN = E // NSUB
CHUNK = 80
MEGA = 10000
NK = MEGA // CHUNK
NM = SPAN // MEGA
ACC_ROWS = 50048
ZROWS = ACC_ROWS // NSUB


NBUF = 6
PRE = 3


def _spmm_body(rows_hbm, cols_hbm, vals_hbm, xs_hbm, zeros_hbm, out_hbm,
               acc, cols_v,
               rows_a, rows_b, rows_c, rows_d, rows_e, rows_f,
               vals_a, vals_b, vals_c, vals_d, vals_e, vals_f,
               gath_a, gath_b, gath_c, gath_d, gath_e, gath_f,
               sem_s, sem_a, sem_b, sem_c, sem_d, sem_e, sem_f,
               sem_sa, sem_sb, sem_sc, sem_sd, sem_se, sem_sf):
    core = lax.axis_index("c")
    sid = lax.axis_index("s")

    RB = (rows_a, rows_b, rows_c, rows_d, rows_e, rows_f)
    VB = (vals_a, vals_b, vals_c, vals_d, vals_e, vals_f)
    GB = (gath_a, gath_b, gath_c, gath_d, gath_e, gath_f)
    SG = (sem_a, sem_b, sem_c, sem_d, sem_e, sem_f)
    SS = (sem_sa, sem_sb, sem_sc, sem_sd, sem_se, sem_sf)

    pltpu.sync_copy(zeros_hbm, acc.at[pl.ds(sid * ZROWS, ZROWS)])

    xsrc = xs_hbm.at[core]

    def start(cb, k, p):
        pltpu.async_copy(rows_hbm.at[cb + k], RB[p], SG[p])
        pltpu.async_copy(vals_hbm.at[pl.ds((cb + k) * CHUNK, CHUNK)],
                         VB[p], SG[p])
        pltpu.async_copy(xsrc.at[cols_v.at[k]], GB[p], SG[p])

    def waitg(cb, k, p):
        pltpu.make_async_copy(rows_hbm.at[cb + k], RB[p], SG[p]).wait()
        pltpu.make_async_copy(vals_hbm.at[pl.ds((cb + k) * CHUNK, CHUNK)],
                              VB[p], SG[p]).wait()
        pltpu.make_async_copy(xsrc.at[cols_v.at[k]], GB[p], SG[p]).wait()

    def compute(k, p):
        gath_buf = GB[p]
        for j in range(CHUNK // 16):
            val16 = VB[p][pl.ds(j * 16, 16)]
            for i in range(16):
                e = j * 16 + i
                vv = lax.gather(
                    val16, jnp.full((16, 1), i, jnp.int32),
                    lax.GatherDimensionNumbers(
                        offset_dims=(), collapsed_slice_dims=(0,),
                        start_index_map=(0,)),
                    slice_sizes=(1,),
                    mode=lax.GatherScatterMode.PROMISE_IN_BOUNDS)
                for c in range(HD // 16):
                    sl = pl.ds(c * 16, 16)
                    gath_buf[e, sl] = gath_buf[e, sl] * vv

    def scat_start(p):
        pltpu.async_copy(GB[p], acc.at[RB[p]], SS[p], add=True)

    def scat_wait(p):
        pltpu.make_async_copy(GB[p], acc.at[RB[p]], SS[p]).wait()

    for p in range(NBUF):
        for e in range(CHUNK):
            for c in range(HD // 16):
                GB[p][e, pl.ds(c * 16, 16)] = jnp.zeros((16,), jnp.float32)
        for e in range(CHUNK // 16):
            RB[p][pl.ds(e * 16, 16)] = jnp.zeros((16,), jnp.int32)
    plsc.subcore_barrier()
    for p in range(NBUF):
        scat_start(p)

    def mega_body(m, _):
        base = sid * SPAN + m * MEGA
        cb = base // CHUNK
        d_cols = pltpu.async_copy(cols_hbm.at[pl.ds(cb, NK)], cols_v, sem_s)
        d_cols.wait()
        for k in range(PRE):
            scat_wait(k % NBUF)
            start(cb, k, k % NBUF)

        def quad_body(t, _):
            k0 = NBUF * t
            for p in range(NBUF):
                k = k0 + p
                waitg(cb, k, p)
                w = (p + PRE) % NBUF
                scat_wait(w)
                start(cb, k + PRE, w)
                compute(k, p)
                scat_start(p)
            return 0

        lax.fori_loop(0, (NK - 5) // NBUF, quad_body, 0)
        for k in range(((NK - 5) // NBUF) * NBUF, NK):
            p = k % NBUF
            waitg(cb, k, p)
            if k + PRE < NK:
                w = (k + PRE) % NBUF
                scat_wait(w)
                start(cb, k + PRE, w)
            compute(k, p)
            scat_start(p)
        return 0

    lax.fori_loop(0, NM, mega_body, 0)
    for p in range(NBUF):
        scat_wait(p)
    plsc.subcore_barrier()

    fl = 3120

    @pl.when(sid < 15)
    def _():
        pltpu.sync_copy(acc.at[pl.ds(sid * fl, fl)],
                        out_hbm.at[core, pl.ds(sid * fl, fl)])

    @pl.when(sid == 15)
    def _():
        pltpu.sync_copy(acc.at[pl.ds(15 * fl, N - 15 * fl)],
                        out_hbm.at[core, pl.ds(15 * fl, N - 15 * fl)])


_spmm_call = pl.kernel(
    _spmm_body,
    out_type=jax.ShapeDtypeStruct((2, N, HD), jnp.float32),
    mesh=plsc.VectorSubcoreMesh(core_axis_name="c", subcore_axis_name="s"),
    compiler_params=pltpu.CompilerParams(
        needs_layout_passes=False, use_tc_tiling_on_sc=False),
    scratch_types=[
        pltpu.VMEM_SHARED((ACC_ROWS, HD), jnp.float32),
        pltpu.VMEM((NK, CHUNK), jnp.int32),
    ] + [pltpu.VMEM((CHUNK,), jnp.int32)] * 6
      + [pltpu.VMEM((CHUNK,), jnp.float32)] * 6
      + [pltpu.VMEM((CHUNK, HD), jnp.float32)] * 6
      + [pltpu.SemaphoreType.DMA] * 13,
)

_ZEROS = None


def _spmm(idx, val, xs):
    global _ZEROS
    if _ZEROS is None:
        _ZEROS = jnp.zeros((ZROWS, HD), jnp.float32)
    rows2 = idx[0].reshape(E // CHUNK, CHUNK)
    cols2 = idx[1].reshape(E // CHUNK, CHUNK)
    return _spmm_call(rows2, cols2, val, xs, _ZEROS)


def _split(x):
    return jnp.stack([x[:, :HD], x[:, HD:]])


_FEAT_BLK = 1000


def _feats_body(x_ref, w_ref, b_ref, o_ref):
    acc = jnp.dot(x_ref[...], w_ref[...], preferred_element_type=jnp.float32)
    acc = acc + b_ref[...]
    n = jnp.sqrt(jnp.sum(acc * acc, axis=1, keepdims=True))
    o_ref[...] = acc / jnp.maximum(n, 1e-12)


def _proj_l2(x, w, b):
    m, k = x.shape
    d = w.shape[1]
    grid = m // _FEAT_BLK
    return pl.pallas_call(
        _feats_body,
        grid=(grid,),
        in_specs=[
            pl.BlockSpec((_FEAT_BLK, k), lambda i: (i, 0)),
            pl.BlockSpec((k, d), lambda i: (0, 0)),
            pl.BlockSpec((1, d), lambda i: (0, 0)),
        ],
        out_specs=pl.BlockSpec((_FEAT_BLK, d), lambda i: (i, 0)),
        out_shape=jax.ShapeDtypeStruct((m, d), jnp.float32),
    )(x, w, b.reshape(1, d))


_FUSE_BLK = 1000


def _modal_body(ia, ib, iadj, ta, tb, tadj, wi, wt, o_ref):
    img = ia[...] + ib[...] + RIS_ADJ_LAMBDA * iadj[...]
    txt = ta[...] + tb[...] + RIS_ADJ_LAMBDA * tadj[...]
    o_ref[...] = wi[...] * img + wt[...] * txt


def _modal_combine(im1, im2, imadj, tx1, tx2, txadj, wi, wt):
    spec = pl.BlockSpec((2, _FUSE_BLK, HD), lambda i: (0, i, 0))
    wspec = pl.BlockSpec((_FUSE_BLK, 1), lambda i: (i, 0))
    return pl.pallas_call(
        _modal_body,
        grid=(N // _FUSE_BLK,),
        in_specs=[spec, spec, spec, spec, spec, spec, wspec, wspec],
        out_specs=spec,
        out_shape=jax.ShapeDtypeStruct((2, N, HD), jnp.float32),
    )(im1, im2, imadj, tx1, tx2, txadj, wi, wt)


def _final_body(m, g1, g2, o_ref):
    m0, m1 = m[0], m[1]
    s = jnp.sum(m0 * m0 + m1 * m1, axis=1, keepdims=True)
    inv = RIS_LAMBDA / jnp.maximum(jnp.sqrt(s), 1e-12)
    o_ref[:, :HD] = m0 + g1[0] + g2[0] + inv * m0
    o_ref[:, HD:] = m1 + g1[1] + g2[1] + inv * m1


def _final_combine(modal, g1, g2):
    spec = pl.BlockSpec((2, _FUSE_BLK, HD), lambda i: (0, i, 0))
    return pl.pallas_call(
        _final_body,
        grid=(N // _FUSE_BLK,),
        in_specs=[spec, spec, spec],
        out_specs=pl.BlockSpec((_FUSE_BLK, LATDIM), lambda i: (i, 0)),
        out_shape=jax.ShapeDtypeStruct((N, LATDIM), jnp.float32),
    )(modal, g1, g2)


def kernel(adj_idx, adj_val, image_adj_idx, image_adj_val, text_adj_idx,
           text_adj_val, att_image_list, att_text_list, uEmbeds, iEmbeds,
           image_embedding, text_embedding, Wi, bi, Wt, bt):
    image_feats_n = _proj_l2(image_embedding, Wi, bi)
    text_feats_n = _proj_l2(text_embedding, Wt, bt)

    u_s = _split(uEmbeds)
    i_s = _split(iEmbeds)
    ui_s = jnp.concatenate([u_s, i_s], axis=1)

    embedsImageAdj = _spmm(image_adj_idx, image_adj_val, ui_s)
    embedsTextAdj = _spmm(text_adj_idx, text_adj_val, ui_s)

    embedsImage1 = _spmm(adj_idx, adj_val,
                         jnp.concatenate([u_s, _split(image_feats_n)], axis=1))
    embedsImage2 = _spmm(adj_idx, adj_val,
                         jnp.concatenate([embedsImage1[:, :USER], i_s], axis=1))
    embedsText1 = _spmm(adj_idx, adj_val,
                        jnp.concatenate([u_s, _split(text_feats_n)], axis=1))
    embedsText2 = _spmm(adj_idx, adj_val,
                        jnp.concatenate([embedsText1[:, :USER], i_s], axis=1))

    weight_sum = att_image_list + att_text_list
    weight_sum = jnp.where(weight_sum == 0, jnp.ones_like(weight_sum), weight_sum)
    wi_att = (att_image_list / weight_sum)[:, None]
    wt_att = (att_text_list / weight_sum)[:, None]

    embedsModal = _modal_combine(embedsImage1, embedsImage2, embedsImageAdj,
                                 embedsText1, embedsText2, embedsTextAdj,
                                 wi_att, wt_att)

    g1 = _spmm(adj_idx, adj_val, embedsModal)
    g2 = _spmm(adj_idx, adj_val, g1)
    embeds = _final_combine(embedsModal, g1, g2)
    return (embeds[:USER], embeds[USER:])
